# baseline (speedup 1.0000x reference)
import functools

import jax
import jax.numpy as jnp
import numpy as np
from jax import lax
from jax.experimental import pallas as pl
from jax.experimental.pallas import tpu as pltpu
from jax.experimental.pallas import tpu_sc as plsc

H = 256
NHEAD = 8
KNN = 30
NLAYERS = 3
BN = 256
NCORES = 2
NSUB = 16
NW = NCORES * NSUB

_f32 = jnp.float32


def _mm(a, b, precision=None):
    return lax.dot_general(a, b, (((1,), (0,)), ((), ())),
                           preferred_element_type=_f32, precision=precision)


def _ln(x, g, b):
    m = jnp.mean(x, axis=1, keepdims=True)
    v = jnp.mean((x - m) ** 2, axis=1, keepdims=True)
    return (x - m) * lax.rsqrt(v + 1e-5) * g + b


def _topk_body(a_ref, bm_ref, sq_ref, d_ref, i_ref):
    b = pl.program_id(0)
    xb = a_ref[...]
    xa = bm_ref[...]
    n = xa.shape[0]
    sq_r = jnp.sum(xb * xb, axis=1, keepdims=True)
    dot = lax.dot_general(xb, xa, (((1,), (1,)), ((), ())),
                          preferred_element_type=_f32)
    d2 = sq_r + sq_ref[...] + dot
    dist = jnp.sqrt(jnp.maximum(d2, 1e-12))
    col = lax.broadcasted_iota(jnp.int32, (BN, n), 1)
    row = b * BN + lax.broadcasted_iota(jnp.int32, (BN, n), 0)
    cur = jnp.where(row == col, dist + 1e6, dist)
    ds, is_ = [], []
    for _ in range(KNN):
        m = jnp.min(cur, axis=1, keepdims=True)
        cand = jnp.where(cur <= m, col, n)
        ik = jnp.min(cand, axis=1, keepdims=True)
        ds.append(m)
        is_.append(ik)
        cur = jnp.where(col == ik, 3e38, cur)
    d_ref[...] = jnp.concatenate(ds, axis=1)
    i_ref[...] = jnp.concatenate(is_, axis=1)


def _topk(A, Bm, SQ):
    n = A.shape[0]
    return pl.pallas_call(
        _topk_body,
        grid=(n // BN,),
        in_specs=[pl.BlockSpec((BN, 16), lambda b: (b, 0)),
                  pl.BlockSpec((n, 16), lambda b: (0, 0)),
                  pl.BlockSpec((1, n), lambda b: (0, 0))],
        out_specs=[pl.BlockSpec((BN, KNN), lambda b: (b, 0)),
                   pl.BlockSpec((BN, KNN), lambda b: (b, 0))],
        out_shape=[jax.ShapeDtypeStruct((n, KNN), _f32),
                   jax.ShapeDtypeStruct((n, KNN), jnp.int32)],
    )(A, Bm, SQ)


def _sc_gather(table, dstidx):
    V, D = table.shape
    B = dstidx.shape[0]
    bpw = B // NW
    nch = 30
    CH = bpw // nch
    mesh = plsc.VectorSubcoreMesh(core_axis_name="c", subcore_axis_name="s")

    @functools.partial(
        pl.kernel, mesh=mesh,
        compiler_params=pltpu.CompilerParams(needs_layout_passes=False),
        out_type=jax.ShapeDtypeStruct((B, D), _f32),
        scratch_types=[pltpu.VMEM((CH,), jnp.int32),
                       pltpu.VMEM((CH, D), _f32),
                       pltpu.SemaphoreType.DMA],
    )
    def k(table_hbm, idx_hbm, out_hbm, idx_v, rows_v, sem):
        wid = lax.axis_index("s") * NCORES + lax.axis_index("c")
        base = wid * bpw

        def body(c, _):
            off = base + c * CH
            pltpu.sync_copy(idx_hbm.at[pl.ds(off, CH)], idx_v)
            pltpu.async_copy(table_hbm.at[idx_v], rows_v, sem).wait()
            pltpu.sync_copy(rows_v, out_hbm.at[pl.ds(off, CH)])
            return 0

        lax.fori_loop(0, nch, body, 0)

    return k(table, dstidx)


_IOTA16 = tuple(range(16))


def _sc_scatter_add(m, dstidx, zacc, V):
    B, D = m.shape
    G = D // 16
    half = B // 2
    CH = 128
    nch = half // CH
    mesh = plsc.VectorSubcoreMesh(core_axis_name="c", subcore_axis_name="s")

    @functools.partial(
        pl.kernel, mesh=mesh,
        compiler_params=pltpu.CompilerParams(needs_layout_passes=False),
        out_type=jax.ShapeDtypeStruct((2, G, V * 16), _f32),
        scratch_types=[pltpu.VMEM((CH,), jnp.int32),
                       pltpu.VMEM((CH * 16,), _f32),
                       pltpu.VMEM((V * 16,), _f32)],
    )
    def k(mt_hbm, idx_hbm, z_hbm, out_hbm, idx_v, buf_v, acc):
        cid = lax.axis_index("c")
        sid = lax.axis_index("s")
        pltpu.sync_copy(z_hbm, acc)
        base = cid * half

        def body(c, _):
            off = base + c * CH
            pltpu.sync_copy(idx_hbm.at[pl.ds(off, CH)], idx_v)
            pltpu.sync_copy(mt_hbm.at[sid, pl.ds(off * 16, CH * 16)], buf_v)
            io = lax.iota(jnp.int32, 16)
            for j in range(CH // 16):
                dv = idx_v[pl.ds(j * 16, 16)]
                for r in range(16):
                    cidx = (io + r) % 16
                    vals = plsc.load_gather(
                        buf_v, [(io + j * 16) * 16 + cidx])
                    plsc.addupdate_scatter(acc, [dv * 16 + cidx], vals)
            return 0

        lax.fori_loop(0, nch, body, 0)
        pltpu.sync_copy(acc, out_hbm.at[cid, sid])

    mt = m.reshape(B, G, 16).transpose(1, 0, 2).reshape(G, B * 16)
    out = k(mt, dstidx, zacc)
    return out.reshape(2, G, V, 16).transpose(0, 2, 1, 3).reshape(2, V, D)


def _sc_counts(dstidx, zacc, V):
    B = dstidx.shape[0]
    half = B // 2
    CH = 512
    nch = half // CH
    mesh = plsc.VectorSubcoreMesh(core_axis_name="c", subcore_axis_name="s")

    @functools.partial(
        pl.kernel, mesh=mesh,
        compiler_params=pltpu.CompilerParams(needs_layout_passes=False),
        out_type=jax.ShapeDtypeStruct((2, V * 16), _f32),
        scratch_types=[pltpu.VMEM((CH,), jnp.int32),
                       pltpu.VMEM((V * 16,), _f32)],
    )
    def k(idx_hbm, z_hbm, out_hbm, idx_v, acc):
        cid = lax.axis_index("c")
        sid = lax.axis_index("s")

        @pl.when(sid == 0)
        def _():
            pltpu.sync_copy(z_hbm, acc)

            def body(c, _):
                off = cid * half + c * CH
                pltpu.sync_copy(idx_hbm.at[pl.ds(off, CH)], idx_v)
                io = lax.iota(jnp.int32, 16)
                ones16 = (io * 0 + 1).astype(_f32)
                for j in range(CH // 16):
                    dv = idx_v[pl.ds(j * 16, 16)]
                    for r in range(16):
                        cidx = (io + r) % 16
                        plsc.addupdate_scatter(acc, [dv * 16 + cidx], ones16)
                return 0

            lax.fori_loop(0, nch, body, 0)
            pltpu.sync_copy(acc, out_hbm.at[cid])

    return k(dstidx, zacc).reshape(2, V, 16)


def _dense_body(x_ref, w_ref, b_ref, o_ref):
    o_ref[...] = _mm(x_ref[...], w_ref[...]) + b_ref[...]


def _dense(x, w, b):
    n, din = x.shape
    dout = w.shape[1]
    return pl.pallas_call(
        _dense_body,
        grid=(n // BN,),
        in_specs=[pl.BlockSpec((BN, din), lambda i: (i, 0)),
                  pl.BlockSpec((din, dout), lambda i: (0, 0)),
                  pl.BlockSpec((1, dout), lambda i: (0, 0))],
        out_specs=pl.BlockSpec((BN, dout), lambda i: (i, 0)),
        out_shape=jax.ShapeDtypeStruct((n, dout), _f32),
    )(x, w, b)


def _hab_body(h_ref, wa_ref, wb_ref, a_ref, b_ref):
    h = h_ref[...]
    a_ref[...] = _mm(h, wa_ref[...])
    b_ref[...] = _mm(h, wb_ref[...])


def _hab(h, wa, wb):
    n = h.shape[0]
    return pl.pallas_call(
        _hab_body,
        grid=(n // BN,),
        in_specs=[pl.BlockSpec((BN, H), lambda i: (i, 0)),
                  pl.BlockSpec((H, H), lambda i: (0, 0)),
                  pl.BlockSpec((H, H), lambda i: (0, 0))],
        out_specs=[pl.BlockSpec((BN, H), lambda i: (i, 0)),
                   pl.BlockSpec((BN, H), lambda i: (i, 0))],
        out_shape=[jax.ShapeDtypeStruct((n, H), _f32),
                   jax.ShapeDtypeStruct((n, H), _f32)],
    )(h, wa, wb)


def _node_body(af_ref, ab_ref, pe_ref, w16_ref, w17_ref, b_ref, o_ref):
    xa = af_ref[...]
    com = jnp.mean(xa, axis=0, keepdims=True)
    dif = ab_ref[...] - com
    dcom = jnp.sqrt(jnp.sum(dif * dif, axis=1, keepdims=True))
    o_ref[...] = (_mm(pe_ref[...], w16_ref[...]) + dcom * w17_ref[...]
                  + b_ref[...])


def _node_embed(A, pe, w16, w17, b):
    n = A.shape[0]
    return pl.pallas_call(
        _node_body,
        grid=(n // BN,),
        in_specs=[pl.BlockSpec((n, 16), lambda i: (0, 0)),
                  pl.BlockSpec((BN, 16), lambda i: (i, 0)),
                  pl.BlockSpec((BN, 16), lambda i: (i, 0)),
                  pl.BlockSpec((16, H), lambda i: (0, 0)),
                  pl.BlockSpec((1, H), lambda i: (0, 0)),
                  pl.BlockSpec((1, H), lambda i: (0, 0))],
        out_specs=pl.BlockSpec((BN, H), lambda i: (i, 0)),
        out_shape=jax.ShapeDtypeStruct((n, H), _f32),
    )(A, A, pe, w16, w17, b)


def _edge_embed_body(d_ref, cd_ref, cs_ref, w_ref, b_ref, o_ref):
    d = d_ref[...]
    mu = lax.broadcasted_iota(jnp.int32, (1, 16), 1).astype(_f32) * (20.0 / 15.0)
    rbf = jnp.exp(-((d - mu) ** 2) * (1.0 / 3.125))
    disp = cd_ref[...] - cs_ref[...]
    nrm = jnp.sqrt(jnp.sum(disp * disp, axis=1, keepdims=True))
    ori = disp / (nrm + 1e-8)
    ef = jnp.concatenate([rbf, ori[:, :16]], axis=1)
    o_ref[...] = _mm(ef, w_ref[...]) + b_ref[...]


def _edge_embed(dcol, cdst, A128, w_pad, b):
    n = A128.shape[0]
    nb = n // BN
    e = dcol.shape[0]
    return pl.pallas_call(
        _edge_embed_body,
        grid=(nb, KNN),
        in_specs=[pl.BlockSpec((BN, 1), lambda i, k: (k * nb + i, 0)),
                  pl.BlockSpec((BN, 128), lambda i, k: (k * nb + i, 0)),
                  pl.BlockSpec((BN, 128), lambda i, k: (i, 0)),
                  pl.BlockSpec((32, H), lambda i, k: (0, 0)),
                  pl.BlockSpec((1, H), lambda i, k: (0, 0))],
        out_specs=pl.BlockSpec((BN, H), lambda i, k: (k * nb + i, 0)),
        out_shape=jax.ShapeDtypeStruct((e, H), _f32),
    )(dcol, cdst, A128, w_pad, b)


def _edge_msg_body(eh_ref, hbg_ref, ha_ref, w1c_ref, b1_ref, o_ref):
    o_ref[...] = jnp.maximum(ha_ref[...] + hbg_ref[...]
                             + _mm(eh_ref[...], w1c_ref[...]) + b1_ref[...],
                             0.0)


def _edge_msg(eh, hbg, ha, w1c, b1):
    e = eh.shape[0]
    n = ha.shape[0]
    nb = n // BN
    return pl.pallas_call(
        _edge_msg_body,
        grid=(nb, KNN),
        in_specs=[pl.BlockSpec((BN, H), lambda i, k: (k * nb + i, 0)),
                  pl.BlockSpec((BN, H), lambda i, k: (k * nb + i, 0)),
                  pl.BlockSpec((BN, H), lambda i, k: (i, 0)),
                  pl.BlockSpec((H, H), lambda i, k: (0, 0)),
                  pl.BlockSpec((1, H), lambda i, k: (0, 0))],
        out_specs=pl.BlockSpec((BN, H), lambda i, k: (k * nb + i, 0)),
        out_shape=jax.ShapeDtypeStruct((e, H), _f32),
    )(eh, hbg, ha, w1c, b1)


def _update_body(h_ref, s0_ref, s1_ref, c0_ref, c1_ref, w2_ref, b2_ref,
                 u1a_ref, u1b_ref, ub1_ref, u2_ref, ub2_ref, g_ref, bt_ref,
                 o_ref):
    h = h_ref[...]
    s = s0_ref[0] + s1_ref[0]
    cnt = c0_ref[0][:, :1] + c1_ref[0][:, :1]
    aggm = _mm(s, w2_ref[...]) + cnt * b2_ref[...]
    t = jnp.maximum(_mm(h, u1a_ref[...]) + _mm(aggm, u1b_ref[...])
                    + ub1_ref[...], 0.0)
    u = _mm(t, u2_ref[...]) + ub2_ref[...]
    o_ref[...] = _ln(h + u, g_ref[...], bt_ref[...])


def _update(h, scat, cnts, w2, b2, u1a, u1b, ub1, u2, ub2, g, bt):
    n = h.shape[0]
    nb = n // BN
    return pl.pallas_call(
        _update_body,
        grid=(nb,),
        in_specs=[pl.BlockSpec((BN, H), lambda i: (i, 0)),
                  pl.BlockSpec((1, BN, H), lambda i: (0, i, 0)),
                  pl.BlockSpec((1, BN, H), lambda i: (1, i, 0)),
                  pl.BlockSpec((1, BN, 16), lambda i: (0, i, 0)),
                  pl.BlockSpec((1, BN, 16), lambda i: (1, i, 0)),
                  pl.BlockSpec((H, H), lambda i: (0, 0)),
                  pl.BlockSpec((1, H), lambda i: (0, 0)),
                  pl.BlockSpec((H, H), lambda i: (0, 0)),
                  pl.BlockSpec((H, H), lambda i: (0, 0)),
                  pl.BlockSpec((1, H), lambda i: (0, 0)),
                  pl.BlockSpec((H, H), lambda i: (0, 0)),
                  pl.BlockSpec((1, H), lambda i: (0, 0)),
                  pl.BlockSpec((1, H), lambda i: (0, 0)),
                  pl.BlockSpec((1, H), lambda i: (0, 0))],
        out_specs=pl.BlockSpec((BN, H), lambda i: (i, 0)),
        out_shape=jax.ShapeDtypeStruct((n, H), _f32),
    )(h, scat, scat, cnts, cnts, w2, b2, u1a, u1b, ub1, u2, ub2, g, bt)


def _attn_body(q_ref, k_ref, v_ref, o_ref):
    q = q_ref[0]
    k = k_ref[0]
    v = v_ref[0]
    s = lax.dot_general(q, k, (((1,), (1,)), ((), ())),
                        preferred_element_type=_f32) * (1.0 / np.sqrt(32.0))
    mx = jnp.max(s, axis=1, keepdims=True)
    p = jnp.exp(s - mx)
    a = p / jnp.sum(p, axis=1, keepdims=True)
    o_ref[0] = _mm(a, v)


def _attn(q3, k3, v3):
    nh, n, dh = q3.shape
    return pl.pallas_call(
        _attn_body,
        grid=(nh, n // BN),
        in_specs=[pl.BlockSpec((1, BN, dh), lambda h, b: (h, b, 0)),
                  pl.BlockSpec((1, n, dh), lambda h, b: (h, 0, 0)),
                  pl.BlockSpec((1, n, dh), lambda h, b: (h, 0, 0))],
        out_specs=pl.BlockSpec((1, BN, dh), lambda h, b: (h, b, 0)),
        out_shape=jax.ShapeDtypeStruct((nh, n, dh), _f32),
    )(q3, k3, v3)


def _dec_tail_body(o_ref, h_ref, wo_ref, bo_ref, g1_ref, b1_ref, f1_ref,
                   fb1_ref, f2_ref, fb2_ref, g2_ref, b2_ref, out_ref):
    a = _mm(o_ref[...], wo_ref[...]) + bo_ref[...]
    x = _ln(h_ref[...] + a, g1_ref[...], b1_ref[...])
    f = _mm(jnp.maximum(_mm(x, f1_ref[...]) + fb1_ref[...], 0.0),
            f2_ref[...]) + fb2_ref[...]
    out_ref[...] = _ln(x + f, g2_ref[...], b2_ref[...])


def _dec_tail(o, h, wo, bo, g1, b1, f1, fb1, f2, fb2, g2, b2):
    n = h.shape[0]
    F = f1.shape[1]
    return pl.pallas_call(
        _dec_tail_body,
        grid=(n // BN,),
        in_specs=[pl.BlockSpec((BN, H), lambda i: (i, 0)),
                  pl.BlockSpec((BN, H), lambda i: (i, 0)),
                  pl.BlockSpec((H, H), lambda i: (0, 0)),
                  pl.BlockSpec((1, H), lambda i: (0, 0)),
                  pl.BlockSpec((1, H), lambda i: (0, 0)),
                  pl.BlockSpec((1, H), lambda i: (0, 0)),
                  pl.BlockSpec((H, F), lambda i: (0, 0)),
                  pl.BlockSpec((1, F), lambda i: (0, 0)),
                  pl.BlockSpec((F, H), lambda i: (0, 0)),
                  pl.BlockSpec((1, H), lambda i: (0, 0)),
                  pl.BlockSpec((1, H), lambda i: (0, 0)),
                  pl.BlockSpec((1, H), lambda i: (0, 0))],
        out_specs=pl.BlockSpec((BN, H), lambda i: (i, 0)),
        out_shape=jax.ShapeDtypeStruct((n, H), _f32),
    )(o, h, wo, bo, g1, b1, f1, fb1, f2, fb2, g2, b2)


def _row(x):
    return x.reshape(1, -1)


def kernel(coords, params):
    n = coords.shape[0]
    e = n * KNN

    sq = jnp.sum(coords * coords, axis=1)
    z13 = jnp.zeros((n, 13), _f32)
    A = jnp.concatenate([coords, z13], axis=1)
    Bm = jnp.concatenate([-2.0 * coords, z13], axis=1)

    dists, idx = _topk(A, Bm, sq.reshape(1, n))
    dst = idx.T.reshape(-1)
    dcol = dists.T.reshape(-1, 1)

    pos = jnp.arange(n, dtype=_f32)
    fr = [pos / 10000.0 ** (2 * i / 16) for i in range(8)]
    pe = jnp.stack([jnp.sin(f) for f in fr] + [jnp.cos(f) for f in fr],
                   axis=-1)

    nw = params['node_W']
    h = _node_embed(A, pe, nw[:16], nw[16:17], _row(params['node_b']))

    A128 = jnp.concatenate([coords, jnp.zeros((n, 125), _f32)], axis=1)
    cdst = _sc_gather(A128, dst)
    ew = params['edge_W']
    ew_pad = jnp.concatenate([ew[:16], ew[16:19], jnp.zeros((13, H), _f32)],
                             axis=0)
    eh = _edge_embed(dcol, cdst, A128, ew_pad, _row(params['edge_b']))

    zacc = jnp.zeros((n * 16,), _f32)
    cnts = _sc_counts(dst, zacc, n)

    for i in range(NLAYERS):
        w1 = params['enc%d_msg_W1' % i]
        ha, hb = _hab(h, w1[:H], w1[H:2 * H])
        hbg = _sc_gather(hb, dst)
        t = _edge_msg(eh, hbg, ha, w1[2 * H:], _row(params['enc%d_msg_b1' % i]))
        scat = _sc_scatter_add(t, dst, zacc, n)
        u1 = params['enc%d_upd_W1' % i]
        h = _update(h, scat, cnts,
                    params['enc%d_msg_W2' % i],
                    _row(params['enc%d_msg_b2' % i]),
                    u1[:H], u1[H:],
                    _row(params['enc%d_upd_b1' % i]),
                    params['enc%d_upd_W2' % i],
                    _row(params['enc%d_upd_b2' % i]),
                    _row(params['enc%d_ln_g' % i]),
                    _row(params['enc%d_ln_b' % i]))

    dh = H // NHEAD
    for i in range(NLAYERS):
        qkv = _dense(h, params['dec%d_qkv_W' % i],
                     _row(params['dec%d_qkv_b' % i]))
        q3 = qkv[:, :H].reshape(n, NHEAD, dh).transpose(1, 0, 2)
        k3 = qkv[:, H:2 * H].reshape(n, NHEAD, dh).transpose(1, 0, 2)
        v3 = qkv[:, 2 * H:].reshape(n, NHEAD, dh).transpose(1, 0, 2)
        o3 = _attn(q3, k3, v3)
        o = o3.transpose(1, 0, 2).reshape(n, H)
        h = _dec_tail(o, h,
                      params['dec%d_out_W' % i], _row(params['dec%d_out_b' % i]),
                      _row(params['dec%d_ln1_g' % i]), _row(params['dec%d_ln1_b' % i]),
                      params['dec%d_ff_W1' % i], _row(params['dec%d_ff_b1' % i]),
                      params['dec%d_ff_W2' % i], _row(params['dec%d_ff_b2' % i]),
                      _row(params['dec%d_ln2_g' % i]), _row(params['dec%d_ln2_b' % i]))

    return _dense(h, params['out_W'], _row(params['out_b']))
